# Initial kernel scaffold; baseline (speedup 1.0000x reference)
import jax
import jax.numpy as jnp
from jax.experimental import pallas as pl


def kernel(char_ids, pad_ids, embedding, pos_embedding, padding_embedding):
    raise NotImplementedError("write your pallas kernel here")


# speedup vs baseline: 6.7782x; 6.7782x over previous
import functools

import jax
import jax.numpy as jnp
from jax import lax
from jax.experimental import pallas as pl
from jax.experimental.pallas import tpu as pltpu
from jax.experimental.pallas import tpu_sc as plsc

B = 4096
S = 200
DIM = 128
N = B * S
NPP = 3 * S

_info = plsc.get_sparse_core_info()
NC, NS, L = _info.num_cores, _info.num_subcores, _info.num_lanes
NW = NC * NS
ROWS_PER_W = N // NW
CH = 1024
G = CH // 128
N_CHUNKS = ROWS_PER_W // CH
IDROWS_PER_CH = CH // 128


def _build_pospad_tc(padding_embedding, pos_embedding):
    def body(pad_ref, pos_ref, out_ref):
        out_ref[...] = pad_ref[...][:, None, :] + pos_ref[0:S][None, :, :]

    return pl.pallas_call(
        body,
        out_shape=jax.ShapeDtypeStruct((3, S, DIM), jnp.float32),
    )(padding_embedding, pos_embedding)


def _sc_lookup(embedding, pospad, char2d, pad2d):
    mesh = plsc.VectorSubcoreMesh(core_axis_name="c", subcore_axis_name="s")

    @functools.partial(
        pl.kernel,
        mesh=mesh,
        out_type=jax.ShapeDtypeStruct((N, DIM), jnp.float32),
        scratch_types=[
            pltpu.VMEM((IDROWS_PER_CH, 128), jnp.int32),
            pltpu.VMEM((IDROWS_PER_CH, 128), jnp.int32),
            pltpu.VMEM((IDROWS_PER_CH, 128), jnp.int32),
            pltpu.VMEM((128, DIM), jnp.float32),
            pltpu.VMEM((128, DIM), jnp.float32),
            pltpu.SemaphoreType.DMA,
            pltpu.SemaphoreType.DMA,
        ],
    )
    def k(emb_hbm, pp_hbm, char_hbm, pad_hbm, out_hbm,
          char_v, pad_v, ppidx_v, bufa, bufb, sem_in, sem_g):
        wid = lax.axis_index("s") * NC + lax.axis_index("c")
        w_row0 = wid * ROWS_PER_W

        def chunk_body(t, carry):
            row0 = pl.multiple_of(w_row0 + t * CH, CH)
            idrow0 = pl.multiple_of(row0 // 128, IDROWS_PER_CH)

            pltpu.sync_copy(char_hbm.at[pl.ds(idrow0, IDROWS_PER_CH)], char_v)
            pltpu.sync_copy(pad_hbm.at[pl.ds(idrow0, IDROWS_PER_CH)], pad_v)

            lane = lax.iota(jnp.int32, L)
            for r in range(IDROWS_PER_CH):
                for c in range(128 // L):
                    base = (row0 + r * 128 + c * L).astype(jnp.int32)
                    s_vec = (base + lane) % S
                    ppidx_v[r, pl.ds(c * L, L)] = (
                        pad_v[r, pl.ds(c * L, L)] * S + s_vec)

            for g in range(G):
                ca = pltpu.async_copy(emb_hbm.at[char_v.at[g]], bufa, sem_g)
                cb = pltpu.async_copy(pp_hbm.at[ppidx_v.at[g]], bufb, sem_g)
                ca.wait()
                cb.wait()

                def add_row(r, _):
                    for c in range(DIM // L):
                        sl = pl.ds(c * L, L)
                        bufa[r, sl] = bufa[r, sl] + bufb[r, sl]
                    return _
                lax.fori_loop(0, 128, add_row, 0)

                pltpu.sync_copy(bufa, out_hbm.at[pl.ds(row0 + g * 128, 128)])
            return carry

        lax.fori_loop(0, N_CHUNKS, chunk_body, 0)

    return k(embedding, pospad, char2d, pad2d)


def kernel(char_ids, pad_ids, embedding, pos_embedding, padding_embedding):
    pospad = _build_pospad_tc(padding_embedding, pos_embedding)
    pospad = pospad.reshape(NPP, DIM)
    char2d = char_ids.reshape(N // 128, 128).astype(jnp.int32)
    pad2d = pad_ids.reshape(N // 128, 128).astype(jnp.int32)
    out = _sc_lookup(embedding, pospad, char2d, pad2d)
    return out.reshape(B, S, DIM)


# speedup vs baseline: 7.5628x; 1.1157x over previous
import functools

import jax
import jax.numpy as jnp
from jax import lax
from jax.experimental import pallas as pl
from jax.experimental.pallas import tpu as pltpu
from jax.experimental.pallas import tpu_sc as plsc

B = 4096
S = 200
DIM = 128
N = B * S
NPP = 3 * S

_info = plsc.get_sparse_core_info()
NC, NS, L = _info.num_cores, _info.num_subcores, _info.num_lanes
NW = NC * NS
ROWS_PER_W = N // NW
CH = 1024
G = CH // 128
N_CHUNKS = ROWS_PER_W // CH
IDROWS_PER_CH = CH // 128


def _build_pospad_tc(padding_embedding, pos_embedding):
    def body(pad_ref, pos_ref, out_ref):
        out_ref[...] = pad_ref[...][:, None, :] + pos_ref[0:S][None, :, :]

    return pl.pallas_call(
        body,
        out_shape=jax.ShapeDtypeStruct((3, S, DIM), jnp.float32),
    )(padding_embedding, pos_embedding)


def _sc_lookup(embedding, pospad, char2d, pad2d):
    mesh = plsc.VectorSubcoreMesh(core_axis_name="c", subcore_axis_name="s")

    @functools.partial(
        pl.kernel,
        mesh=mesh,
        out_type=jax.ShapeDtypeStruct((N, DIM), jnp.float32),
        scratch_types=[
            pltpu.VMEM((IDROWS_PER_CH, 128), jnp.int32),
            pltpu.VMEM((IDROWS_PER_CH, 128), jnp.int32),
            pltpu.VMEM((IDROWS_PER_CH, 128), jnp.int32),
            pltpu.VMEM((2, 128, DIM), jnp.float32),
            pltpu.VMEM((2, 128, DIM), jnp.float32),
            pltpu.SemaphoreType.DMA,
            pltpu.SemaphoreType.DMA,
        ],
    )
    def k(emb_hbm, pp_hbm, char_hbm, pad_hbm, out_hbm,
          char_v, pad_v, ppidx_v, bufa, bufb, sem_g, sem_o):
        wid = lax.axis_index("s") * NC + lax.axis_index("c")
        w_row0 = wid * ROWS_PER_W

        def chunk_body(t, carry):
            row0 = pl.multiple_of(w_row0 + t * CH, CH)
            idrow0 = pl.multiple_of(row0 // 128, IDROWS_PER_CH)

            pltpu.sync_copy(char_hbm.at[pl.ds(idrow0, IDROWS_PER_CH)], char_v)
            pltpu.sync_copy(pad_hbm.at[pl.ds(idrow0, IDROWS_PER_CH)], pad_v)

            lane = lax.iota(jnp.int32, L)
            for r in range(IDROWS_PER_CH):
                for c in range(128 // L):
                    base = (row0 + r * 128 + c * L).astype(jnp.int32)
                    s_vec = (base + lane) % S
                    ppidx_v[r, pl.ds(c * L, L)] = (
                        pad_v[r, pl.ds(c * L, L)] * S + s_vec)

            gath = [None] * G
            outc = [None] * G
            for g in range(G + 1):
                if g < G:
                    if g >= 2:
                        outc[g - 2].wait()
                    sl = g % 2
                    gath[g] = (
                        pltpu.async_copy(emb_hbm.at[char_v.at[g]],
                                         bufa.at[sl], sem_g),
                        pltpu.async_copy(pp_hbm.at[ppidx_v.at[g]],
                                         bufb.at[sl], sem_g),
                    )
                if g > 0:
                    p = g - 1
                    sp = p % 2
                    gath[p][0].wait()
                    gath[p][1].wait()

                    def add_row(r, _, sp=sp):
                        for c in range(DIM // L):
                            cs = pl.ds(c * L, L)
                            bufa[sp, r, cs] = bufa[sp, r, cs] + bufb[sp, r, cs]
                        return _
                    lax.fori_loop(0, 128, add_row, 0)

                    outc[p] = pltpu.async_copy(
                        bufa.at[sp], out_hbm.at[pl.ds(row0 + p * 128, 128)],
                        sem_o)
            outc[G - 2].wait()
            outc[G - 1].wait()
            return carry

        lax.fori_loop(0, N_CHUNKS, chunk_body, 0)

    return k(embedding, pospad, char2d, pad2d)


def kernel(char_ids, pad_ids, embedding, pos_embedding, padding_embedding):
    pospad = _build_pospad_tc(padding_embedding, pos_embedding)
    pospad = pospad.reshape(NPP, DIM)
    char2d = char_ids.reshape(N // 128, 128).astype(jnp.int32)
    pad2d = pad_ids.reshape(N // 128, 128).astype(jnp.int32)
    out = _sc_lookup(embedding, pospad, char2d, pad2d)
    return out.reshape(B, S, DIM)
